# Initial kernel scaffold; baseline (speedup 1.0000x reference)
import jax
import jax.numpy as jnp
from jax.experimental import pallas as pl


def kernel(x, score):
    raise NotImplementedError("write your pallas kernel here")


# speedup vs baseline: 85.5328x; 85.5328x over previous
import functools

import jax
import jax.numpy as jnp
from jax import lax
from jax.experimental import pallas as pl
from jax.experimental.pallas import tpu as pltpu
from jax.experimental.pallas import tpu_sc as plsc

BLOCK = 8
KEEP = 4
N = 4096 * 4096
NC = 2
NS = 16
L = 16
NW = NC * NS
PER_W = N // NW
CH = 16384
GROUPS = CH // (BLOCK * L)
OUTER = PER_W // CH

_mesh = plsc.VectorSubcoreMesh(core_axis_name="c", subcore_axis_name="s")


@functools.partial(
    pl.kernel,
    out_type=jax.ShapeDtypeStruct((N,), jnp.float32),
    mesh=_mesh,
    scratch_types=[
        pltpu.VMEM((CH,), jnp.float32),
        pltpu.VMEM((CH,), jnp.float32),
        pltpu.VMEM((CH,), jnp.float32),
    ],
    compiler_params=pltpu.CompilerParams(needs_layout_passes=False),
)
def _sparsify_sc(x_hbm, s_hbm, o_hbm, xbuf, sbuf, obuf):
    wid = lax.axis_index("s") * NC + lax.axis_index("c")
    base0 = wid * PER_W
    lane = lax.iota(jnp.int32, L)
    offs = [lane * BLOCK + j for j in range(BLOCK)]
    one = jnp.full((L,), 1, jnp.int32)
    zero = jnp.full((L,), 0, jnp.int32)
    thresh = jnp.full((L,), BLOCK - KEEP, jnp.int32)
    zerof = jnp.full((L,), 0.0, jnp.float32)

    def outer(i, carry):
        off = pl.multiple_of(base0 + i * CH, 8)
        pltpu.sync_copy(s_hbm.at[pl.ds(off, CH)], sbuf)
        pltpu.sync_copy(x_hbm.at[pl.ds(off, CH)], xbuf)

        def inner(g, icarry):
            b = g * (BLOCK * L)
            idx = [b + offs[j] for j in range(BLOCK)]
            s = [plsc.load_gather(sbuf, [idx[j]]) for j in range(BLOCK)]
            r = [jnp.full((L,), BLOCK - 1 - j, jnp.int32) for j in range(BLOCK)]
            for j in range(BLOCK):
                for k in range(j):
                    c = jnp.where(s[k] <= s[j], one, zero)
                    r[j] = r[j] + c
                    r[k] = r[k] - c
            for j in range(BLOCK):
                xv = plsc.load_gather(xbuf, [idx[j]])
                ov = jnp.where(r[j] >= thresh, xv, zerof)
                plsc.store_scatter(obuf, [idx[j]], ov)
            return icarry

        lax.fori_loop(0, GROUPS, inner, None)
        pltpu.sync_copy(obuf, o_hbm.at[pl.ds(off, CH)])
        return carry

    lax.fori_loop(0, OUTER, outer, None)


def kernel(x, score):
    out = _sparsify_sc(x.reshape(-1), score.reshape(-1))
    return out.reshape(x.shape)


# speedup vs baseline: 106.6881x; 1.2473x over previous
import functools

import jax
import jax.numpy as jnp
from jax import lax
from jax.experimental import pallas as pl
from jax.experimental.pallas import tpu as pltpu
from jax.experimental.pallas import tpu_sc as plsc

BLOCK = 8
KEEP = 4
N = 4096 * 4096
NC = 2
NS = 16
L = 16
NW = NC * NS
PER_W = N // NW
CH = 16384
GROUPS = CH // (BLOCK * L)
OUTER = PER_W // CH

_mesh = plsc.VectorSubcoreMesh(core_axis_name="c", subcore_axis_name="s")


@functools.partial(
    pl.kernel,
    out_type=jax.ShapeDtypeStruct((N,), jnp.float32),
    mesh=_mesh,
    scratch_types=[
        pltpu.VMEM((CH,), jnp.float32),
        pltpu.VMEM((CH,), jnp.float32),
        pltpu.VMEM((CH,), jnp.float32),
        pltpu.VMEM((CH,), jnp.float32),
        pltpu.VMEM((CH,), jnp.float32),
        pltpu.VMEM((CH,), jnp.float32),
        pltpu.SemaphoreType.DMA,
        pltpu.SemaphoreType.DMA,
        pltpu.SemaphoreType.DMA,
        pltpu.SemaphoreType.DMA,
        pltpu.SemaphoreType.DMA,
        pltpu.SemaphoreType.DMA,
    ],
    compiler_params=pltpu.CompilerParams(needs_layout_passes=False),
)
def _sparsify_sc(x_hbm, s_hbm, o_hbm,
                 sA, sB, xA, xB, oA, oB,
                 ssA, ssB, sxA, sxB, soA, soB):
    wid = lax.axis_index("s") * NC + lax.axis_index("c")
    base0 = wid * PER_W
    lane = lax.iota(jnp.int32, L)
    offs = [lane * BLOCK + j for j in range(BLOCK)]
    one = jnp.full((L,), 1, jnp.int32)
    zero = jnp.full((L,), 0, jnp.int32)
    thresh = jnp.full((L,), BLOCK - KEEP, jnp.int32)
    zerof = jnp.full((L,), 0.0, jnp.float32)

    slots = ((sA, xA, oA, ssA, sxA, soA), (sB, xB, oB, ssB, sxB, soB))

    def compute_chunk(sbuf, xbuf, obuf):
        def inner(g, icarry):
            b = g * (BLOCK * L)
            idx = [b + offs[j] for j in range(BLOCK)]
            s = [plsc.load_gather(sbuf, [idx[j]]) for j in range(BLOCK)]
            r = [jnp.full((L,), BLOCK - 1 - j, jnp.int32) for j in range(BLOCK)]
            for j in range(BLOCK):
                for k in range(j):
                    c = jnp.where(s[k] <= s[j], one, zero)
                    r[j] = r[j] + c
                    r[k] = r[k] - c
            for j in range(BLOCK):
                xv = plsc.load_gather(xbuf, [idx[j]])
                ov = jnp.where(r[j] >= thresh, xv, zerof)
                plsc.store_scatter(obuf, [idx[j]], ov)
            return icarry

        lax.fori_loop(0, GROUPS, inner, None)

    def start_in(i, sbuf, xbuf, ssem, xsem):
        off = pl.multiple_of(base0 + i * CH, 8)
        pltpu.async_copy(s_hbm.at[pl.ds(off, CH)], sbuf, ssem)
        pltpu.async_copy(x_hbm.at[pl.ds(off, CH)], xbuf, xsem)

    for b in range(2):
        sbuf, xbuf, obuf, ssem, xsem, osem = slots[b]
        start_in(b, sbuf, xbuf, ssem, xsem)

    def outer(io, carry):
        for b in range(2):
            i = io * 2 + b
            sbuf, xbuf, obuf, ssem, xsem, osem = slots[b]
            off = pl.multiple_of(base0 + i * CH, 8)
            pltpu.make_async_copy(s_hbm.at[pl.ds(0, CH)], sbuf, ssem).wait()
            pltpu.make_async_copy(x_hbm.at[pl.ds(0, CH)], xbuf, xsem).wait()
            @pl.when(i >= 2)
            def _():
                pltpu.make_async_copy(obuf, o_hbm.at[pl.ds(0, CH)], osem).wait()

            compute_chunk(sbuf, xbuf, obuf)
            pltpu.async_copy(obuf, o_hbm.at[pl.ds(off, CH)], osem)

            @pl.when(i + 2 < OUTER)
            def _():
                start_in(i + 2, sbuf, xbuf, ssem, xsem)
        return carry

    lax.fori_loop(0, OUTER // 2, outer, None)

    for b in range(2):
        sbuf, xbuf, obuf, ssem, xsem, osem = slots[b]
        pltpu.make_async_copy(obuf, o_hbm.at[pl.ds(0, CH)], osem).wait()


def kernel(x, score):
    out = _sparsify_sc(x.reshape(-1), score.reshape(-1))
    return out.reshape(x.shape)


# speedup vs baseline: 169.4992x; 1.5887x over previous
import functools

import jax
import jax.numpy as jnp
from jax import lax
from jax.experimental import pallas as pl
from jax.experimental.pallas import tpu as pltpu
from jax.experimental.pallas import tpu_sc as plsc

BLOCK = 8
KEEP = 4
NROW = 4096
NCOL = 4096
NC = 2
NS = 16
L = 16
NW = NC * NS
ROWS_W = NROW // NW
RCH = 4
OUTER = ROWS_W // RCH
GROUPS = NCOL // (BLOCK * L)

_mesh = plsc.VectorSubcoreMesh(core_axis_name="c", subcore_axis_name="s")


@functools.partial(
    pl.kernel,
    out_type=jax.ShapeDtypeStruct((NROW, NCOL), jnp.float32),
    mesh=_mesh,
    scratch_types=[
        pltpu.VMEM((RCH, NCOL), jnp.float32),
        pltpu.VMEM((RCH, NCOL), jnp.float32),
        pltpu.VMEM((RCH, NCOL), jnp.float32),
        pltpu.VMEM((RCH, NCOL), jnp.float32),
        pltpu.VMEM((RCH, NCOL), jnp.float32),
        pltpu.VMEM((RCH, NCOL), jnp.float32),
        pltpu.SemaphoreType.DMA,
        pltpu.SemaphoreType.DMA,
        pltpu.SemaphoreType.DMA,
        pltpu.SemaphoreType.DMA,
        pltpu.SemaphoreType.DMA,
        pltpu.SemaphoreType.DMA,
    ],
    compiler_params=pltpu.CompilerParams(needs_layout_passes=False),
)
def _sparsify_sc(x_hbm, s_hbm, o_hbm,
                 sA, sB, xA, xB, oA, oB,
                 ssA, ssB, sxA, sxB, soA, soB):
    wid = lax.axis_index("s") * NC + lax.axis_index("c")
    row0 = wid * ROWS_W
    lane = lax.iota(jnp.int32, L)
    offs = [lane * BLOCK + j for j in range(BLOCK)]
    one = jnp.full((L,), 1, jnp.int32)
    zero = jnp.full((L,), 0, jnp.int32)
    thresh = jnp.full((L,), BLOCK - KEEP, jnp.int32)
    zerof = jnp.full((L,), 0.0, jnp.float32)

    slots = ((sA, xA, oA, ssA, sxA, soA), (sB, xB, oB, ssB, sxB, soB))

    def compute_chunk(sbuf, xbuf, obuf):
        for rr in range(RCH):
            rowv = jnp.full((L,), rr, jnp.int32)

            def inner(g, icarry):
                cb = g * (BLOCK * L)
                idx = [cb + offs[j] for j in range(BLOCK)]
                s = [plsc.load_gather(sbuf, [rowv, idx[j]])
                     for j in range(BLOCK)]
                r = [jnp.full((L,), BLOCK - 1 - j, jnp.int32)
                     for j in range(BLOCK)]
                for j in range(BLOCK):
                    for k in range(j):
                        c = jnp.where(s[k] <= s[j], one, zero)
                        r[j] = r[j] + c
                        r[k] = r[k] - c
                for j in range(BLOCK):
                    xv = plsc.load_gather(xbuf, [rowv, idx[j]])
                    ov = jnp.where(r[j] >= thresh, xv, zerof)
                    plsc.store_scatter(obuf, [rowv, idx[j]], ov)
                return icarry

            lax.fori_loop(0, GROUPS, inner, None)

    def start_in(i, sbuf, xbuf, ssem, xsem):
        r = row0 + i * RCH
        pltpu.async_copy(s_hbm.at[pl.ds(r, RCH)], sbuf, ssem)
        pltpu.async_copy(x_hbm.at[pl.ds(r, RCH)], xbuf, xsem)

    for b in range(2):
        sbuf, xbuf, obuf, ssem, xsem, osem = slots[b]
        start_in(b, sbuf, xbuf, ssem, xsem)

    def outer(io, carry):
        for b in range(2):
            i = io * 2 + b
            sbuf, xbuf, obuf, ssem, xsem, osem = slots[b]
            r = row0 + i * RCH
            pltpu.make_async_copy(s_hbm.at[pl.ds(0, RCH)], sbuf, ssem).wait()
            pltpu.make_async_copy(x_hbm.at[pl.ds(0, RCH)], xbuf, xsem).wait()
            @pl.when(i >= 2)
            def _():
                pltpu.make_async_copy(obuf, o_hbm.at[pl.ds(0, RCH)],
                                      osem).wait()

            compute_chunk(sbuf, xbuf, obuf)
            pltpu.async_copy(obuf, o_hbm.at[pl.ds(r, RCH)], osem)

            @pl.when(i + 2 < OUTER)
            def _():
                start_in(i + 2, sbuf, xbuf, ssem, xsem)
        return carry

    lax.fori_loop(0, OUTER // 2, outer, None)

    for b in range(2):
        sbuf, xbuf, obuf, ssem, xsem, osem = slots[b]
        pltpu.make_async_copy(obuf, o_hbm.at[pl.ds(0, RCH)], osem).wait()


def kernel(x, score):
    return _sparsify_sc(x, score)


# speedup vs baseline: 384.9019x; 2.2708x over previous
import functools

import jax
import jax.numpy as jnp
from jax import lax
from jax.experimental import pallas as pl
from jax.experimental.pallas import tpu as pltpu
from jax.experimental.pallas import tpu_sc as plsc

BLOCK = 8
KEEP = 4
NROW = 4096
NCOL = 4096
NC = 2
NS = 16
L = 16
NW = NC * NS
ROWS_W = NROW // NW
RCH = 4
OUTER = ROWS_W // RCH
GROUPS = NCOL // (BLOCK * L)

_mesh = plsc.VectorSubcoreMesh(core_axis_name="c", subcore_axis_name="s")


@functools.partial(
    pl.kernel,
    out_type=jax.ShapeDtypeStruct((NROW, NCOL), jnp.float32),
    mesh=_mesh,
    scratch_types=[
        pltpu.VMEM((RCH, NCOL), jnp.float32),
        pltpu.VMEM((RCH, NCOL), jnp.float32),
        pltpu.VMEM((RCH, NCOL), jnp.float32),
        pltpu.VMEM((RCH, NCOL), jnp.float32),
        pltpu.VMEM((RCH, NCOL), jnp.float32),
        pltpu.VMEM((RCH, NCOL), jnp.float32),
        pltpu.SemaphoreType.DMA,
        pltpu.SemaphoreType.DMA,
        pltpu.SemaphoreType.DMA,
        pltpu.SemaphoreType.DMA,
        pltpu.SemaphoreType.DMA,
        pltpu.SemaphoreType.DMA,
    ],
    compiler_params=pltpu.CompilerParams(needs_layout_passes=False),
)
def _sparsify_sc(x_hbm, s_hbm, o_hbm,
                 sA, sB, xA, xB, oA, oB,
                 ssA, ssB, sxA, sxB, soA, soB):
    wid = lax.axis_index("s") * NC + lax.axis_index("c")
    row0 = wid * ROWS_W
    lane = lax.iota(jnp.int32, L)
    offs = tuple(lane * BLOCK + j for j in range(BLOCK))
    step = jnp.full((L,), BLOCK * L, jnp.int32)
    zerof = jnp.full((L,), 0.0, jnp.float32)

    slots = ((sA, xA, oA, ssA, sxA, soA), (sB, xB, oB, ssB, sxB, soB))

    def ce(a, b):
        return jnp.minimum(a, b), jnp.maximum(a, b)

    def sort4(a, b, c, d):
        a, b = ce(a, b)
        c, d = ce(c, d)
        a, c = ce(a, c)
        b, d = ce(b, d)
        b, c = ce(b, c)
        return a, b, c, d

    def compute_chunk(sbuf, xbuf, obuf):
        for rr in range(RCH):
            rowv = jnp.full((L,), rr, jnp.int32)

            def inner(g, idx):
                s = [plsc.load_gather(sbuf, [rowv, idx[j]])
                     for j in range(BLOCK)]
                x = [plsc.load_gather(xbuf, [rowv, idx[j]])
                     for j in range(BLOCK)]
                a = sort4(s[0], s[1], s[2], s[3])
                b = sort4(s[4], s[5], s[6], s[7])
                hi = [jnp.maximum(a[i], b[3 - i]) for i in range(4)]
                t = jnp.minimum(jnp.minimum(hi[0], hi[1]),
                                jnp.minimum(hi[2], hi[3]))
                for j in range(BLOCK):
                    ov = jnp.where(s[j] >= t, x[j], zerof)
                    plsc.store_scatter(obuf, [rowv, idx[j]], ov)
                return tuple(idx[j] + step for j in range(BLOCK))

            lax.fori_loop(0, GROUPS, inner, offs)

    def start_in(i, sbuf, xbuf, ssem, xsem):
        r = row0 + i * RCH
        pltpu.async_copy(s_hbm.at[pl.ds(r, RCH)], sbuf, ssem)
        pltpu.async_copy(x_hbm.at[pl.ds(r, RCH)], xbuf, xsem)

    for b in range(2):
        sbuf, xbuf, obuf, ssem, xsem, osem = slots[b]
        start_in(b, sbuf, xbuf, ssem, xsem)

    def outer(io, carry):
        for b in range(2):
            i = io * 2 + b
            sbuf, xbuf, obuf, ssem, xsem, osem = slots[b]
            r = row0 + i * RCH
            pltpu.make_async_copy(s_hbm.at[pl.ds(0, RCH)], sbuf, ssem).wait()
            pltpu.make_async_copy(x_hbm.at[pl.ds(0, RCH)], xbuf, xsem).wait()
            @pl.when(i >= 2)
            def _():
                pltpu.make_async_copy(obuf, o_hbm.at[pl.ds(0, RCH)],
                                      osem).wait()

            compute_chunk(sbuf, xbuf, obuf)
            pltpu.async_copy(obuf, o_hbm.at[pl.ds(r, RCH)], osem)

            @pl.when(i + 2 < OUTER)
            def _():
                start_in(i + 2, sbuf, xbuf, ssem, xsem)
        return carry

    lax.fori_loop(0, OUTER // 2, outer, None)

    for b in range(2):
        sbuf, xbuf, obuf, ssem, xsem, osem = slots[b]
        pltpu.make_async_copy(obuf, o_hbm.at[pl.ds(0, RCH)], osem).wait()


def kernel(x, score):
    return _sparsify_sc(x, score)
